# baseline (speedup 1.0000x reference)
import functools

import jax
import jax.numpy as jnp
from jax import lax
from jax.experimental import pallas as pl
from jax.experimental.pallas import tpu as pltpu
from jax.experimental.pallas import tpu_sc as plsc

N = 10000
E = 160000
D = 256
DE = 16
H = 128

NT = 16
EPT = E // NT
C = 80
CH = EPT // C
RPB = 624

C2 = 128
NCH2 = E // C2
CH2 = NCH2 // 32

_NODE_BLK = 2000
_EDGE_BLK = 4000


def _tc_node_pre_body(h_ref, phT_ref, wuT_ref, wvT_ref, hp_ref, wu_ref, wv_ref):
    hblk = h_ref[...]
    hp = jnp.dot(hblk, phT_ref[...], preferred_element_type=jnp.float32)
    hp_ref[0] = hp[:, :H]
    hp_ref[1] = hp[:, H:]
    wu_ref[...] = jnp.dot(hblk, wuT_ref[...], preferred_element_type=jnp.float32)
    wv_ref[...] = jnp.dot(hblk, wvT_ref[...], preferred_element_type=jnp.float32)


def _tc_node_pre(h, phT, wuT, wvT):
    nb = N // _NODE_BLK
    return pl.pallas_call(
        _tc_node_pre_body,
        grid=(nb,),
        in_specs=[
            pl.BlockSpec((_NODE_BLK, D), lambda i: (i, 0)),
            pl.BlockSpec((D, D), lambda i: (0, 0)),
            pl.BlockSpec((D, DE), lambda i: (0, 0)),
            pl.BlockSpec((D, DE), lambda i: (0, 0)),
        ],
        out_specs=[
            pl.BlockSpec((2, _NODE_BLK, H), lambda i: (0, i, 0)),
            pl.BlockSpec((_NODE_BLK, DE), lambda i: (i, 0)),
            pl.BlockSpec((_NODE_BLK, DE), lambda i: (i, 0)),
        ],
        out_shape=[
            jax.ShapeDtypeStruct((2, N, H), jnp.float32),
            jax.ShapeDtypeStruct((N, DE), jnp.float32),
            jax.ShapeDtypeStruct((N, DE), jnp.float32),
        ],
    )(h, phT, wuT, wvT)


def _tc_edge_pre_body(e_ref, peT_ref, pb_ref, weT_ref, wb_ref, ep_ref, ew_ref):
    eblk = e_ref[...]
    ep = (jnp.dot(eblk, peT_ref[...], preferred_element_type=jnp.float32)
          + pb_ref[...])
    ep_ref[0] = ep[:, :H]
    ep_ref[1] = ep[:, H:]
    ew_ref[...] = (jnp.dot(eblk, weT_ref[...],
                           preferred_element_type=jnp.float32) + wb_ref[...])


def _tc_edge_pre(e, peT, pb, weT, wb):
    nb = E // _EDGE_BLK
    return pl.pallas_call(
        _tc_edge_pre_body,
        grid=(nb,),
        in_specs=[
            pl.BlockSpec((_EDGE_BLK, DE), lambda i: (i, 0)),
            pl.BlockSpec((DE, D), lambda i: (0, 0)),
            pl.BlockSpec((1, D), lambda i: (0, 0)),
            pl.BlockSpec((DE, DE), lambda i: (0, 0)),
            pl.BlockSpec((1, DE), lambda i: (0, 0)),
        ],
        out_specs=[
            pl.BlockSpec((2, _EDGE_BLK, H), lambda i: (0, i, 0)),
            pl.BlockSpec((_EDGE_BLK, DE), lambda i: (i, 0)),
        ],
        out_shape=[
            jax.ShapeDtypeStruct((2, E, H), jnp.float32),
            jax.ShapeDtypeStruct((E, DE), jnp.float32),
        ],
    )(e, peT, pb, weT, wb)


def _tc_node_out_body(h_ref, a0_ref, a1_ref, d0_ref, d1_ref, qhT_ref, qaT_ref,
                      qb_ref, out_ref):
    agg = jnp.concatenate([a0_ref[...], a1_ref[...]], axis=-1)
    deg = d0_ref[:, 0:1] + d1_ref[:, 0:1]
    degc = jnp.maximum(deg, 1.0)
    aggn = agg / degc
    acc = (jnp.dot(h_ref[...], qhT_ref[...], preferred_element_type=jnp.float32)
           + jnp.dot(aggn, qaT_ref[...], preferred_element_type=jnp.float32)
           + qb_ref[...])
    out_ref[...] = jnp.maximum(acc, 0.0)


def _tc_node_out(h, agg, deg2, qhT, qaT, qb):
    nb = N // _NODE_BLK
    return pl.pallas_call(
        _tc_node_out_body,
        grid=(nb,),
        in_specs=[
            pl.BlockSpec((_NODE_BLK, D), lambda i: (i, 0)),
            pl.BlockSpec((_NODE_BLK, H), lambda i: (i, 0)),
            pl.BlockSpec((_NODE_BLK, H), lambda i, nb=nb: (i + nb, 0)),
            pl.BlockSpec((_NODE_BLK, DE), lambda i: (i, 0)),
            pl.BlockSpec((_NODE_BLK, DE), lambda i, nb=nb: (i + nb, 0)),
            pl.BlockSpec((D, D), lambda i: (0, 0)),
            pl.BlockSpec((D, D), lambda i: (0, 0)),
            pl.BlockSpec((1, D), lambda i: (0, 0)),
        ],
        out_specs=pl.BlockSpec((_NODE_BLK, D), lambda i: (i, 0)),
        out_shape=jax.ShapeDtypeStruct((N, D), jnp.float32),
    )(h, agg, agg, deg2, deg2, qhT, qaT, qb)


def _sc1_body(hp_hbm, ep_hbm, eidx_hbm, agg_hbm,
              agg_sh,
              idx20, src20, hrows0, erows0, sem0, isem0,
              idx21, src21, hrows1, erows1, sem1, isem1,
              zbuf):
    c = lax.axis_index("c")
    s = lax.axis_index("s")
    zero16 = jnp.zeros((16,), jnp.float32)

    def zb_body(i, carry):
        for j in range(H // 16):
            zbuf[i, pl.ds(j * 16, 16)] = zero16
        return carry
    lax.fori_loop(0, 16, zb_body, 0)

    row0 = s * RPB
    ncopy = jnp.where(s == NT - 1, (RPB + 16) // 16, RPB // 16)

    def zinit(i, carry):
        pltpu.sync_copy(zbuf, agg_sh.at[pl.ds(row0 + i * 16, 16)])
        return carry
    lax.fori_loop(0, ncopy, zinit, 0)
    plsc.subcore_barrier()

    tile_base = s * EPT
    hp_off = c * N
    bufs = ((idx20, src20, hrows0, erows0, sem0, isem0),
            (idx21, src21, hrows1, erows1, sem1, isem1))

    def prefetch(k, b):
        idx2_v, _, _, _, _, isem = b
        pltpu.async_copy(eidx_hbm.at[s * CH + k], idx2_v, isem)

    def fire(k, b):
        idx2_v, src2_v, hr, er, sem, isem = b
        base = tile_base + k * C
        pltpu.make_async_copy(eidx_hbm.at[s * CH + k], idx2_v, isem).wait()
        for j in range(C // 16):
            sl = pl.ds(j * 16, 16)
            src2_v[sl] = idx2_v[0, sl] + hp_off
        pltpu.async_copy(hp_hbm.at[src2_v], hr, sem)
        pltpu.async_copy(ep_hbm.at[pl.ds(c * E + base, C)], er, sem)

    def consume(k, b):
        idx2_v, src2_v, hr, er, sem, isem = b
        base = tile_base + k * C
        pltpu.make_async_copy(hp_hbm.at[src2_v], hr, sem).wait()
        pltpu.make_async_copy(ep_hbm.at[pl.ds(c * E + base, C)], er, sem).wait()

        def row(i, carry2):
            for jj in range(4):
                for j in range(H // 16):
                    sl = pl.ds(j * 16, 16)
                    hr[4 * i + jj, sl] = jnp.maximum(
                        hr[4 * i + jj, sl] + er[4 * i + jj, sl], 0.0)
            return carry2
        lax.fori_loop(0, C // 4, row, 0)
        pltpu.sync_copy(hr, agg_sh.at[idx2_v.at[1]], add=True)

    prefetch(0, bufs[0])
    prefetch(1, bufs[1])
    fire(0, bufs[0])
    PAIRS = (CH - 1) // 2

    def pair(i, carry):
        k0 = 2 * i
        fire(k0 + 1, bufs[1])
        consume(k0, bufs[0])
        prefetch(k0 + 2, bufs[0])

        @pl.when(i < PAIRS - 1)
        def _():
            fire(k0 + 2, bufs[0])
        consume(k0 + 1, bufs[1])

        @pl.when(i < PAIRS - 1)
        def _():
            prefetch(k0 + 3, bufs[1])
        return carry
    lax.fori_loop(0, PAIRS, pair, 0)

    for k in range(2 * PAIRS, CH):
        fire(k, bufs[k % 2])
        consume(k, bufs[k % 2])
    plsc.subcore_barrier()

    @pl.when(s < NT - 1)
    def _():
        pltpu.sync_copy(agg_sh.at[pl.ds(row0, RPB)],
                        agg_hbm.at[pl.ds(c * N + row0, RPB)])

    @pl.when(s == NT - 1)
    def _():
        last0 = (NT - 1) * RPB
        nlast = N - last0
        pltpu.sync_copy(agg_sh.at[pl.ds(last0, nlast)],
                        agg_hbm.at[pl.ds(c * N + last0, nlast)])


_sc_aggregate = functools.partial(
    pl.kernel,
    out_type=jax.ShapeDtypeStruct((2 * N, H), jnp.float32),
    mesh=plsc.VectorSubcoreMesh(core_axis_name="c", subcore_axis_name="s"),
    compiler_params=pltpu.CompilerParams(use_tc_tiling_on_sc=False),
    scratch_types=(
        pltpu.VMEM_SHARED((N, H), jnp.float32),
        pltpu.VMEM((2, C), jnp.int32),
        pltpu.VMEM((C,), jnp.int32),
        pltpu.VMEM((C, H), jnp.float32),
        pltpu.VMEM((C, H), jnp.float32),
        pltpu.SemaphoreType.DMA,
        pltpu.SemaphoreType.DMA,
        pltpu.VMEM((2, C), jnp.int32),
        pltpu.VMEM((C,), jnp.int32),
        pltpu.VMEM((C, H), jnp.float32),
        pltpu.VMEM((C, H), jnp.float32),
        pltpu.SemaphoreType.DMA,
        pltpu.SemaphoreType.DMA,
        pltpu.VMEM((16, H), jnp.float32),
    ),
)(_sc1_body)


def _sc2_body(eidx_hbm, wu_hbm, wv_hbm, ew_hbm,
              enew_hbm, deg_hbm,
              deg_sh,
              idx20, urows0, vrows0, wrows0, sem0, isem0,
              idx21, urows1, vrows1, wrows1, sem1, isem1,
              ones_v, zdeg):
    c = lax.axis_index("c")
    s = lax.axis_index("s")
    zero16 = jnp.zeros((16,), jnp.float32)
    one16 = jnp.full((16,), 1.0, jnp.float32)

    def zd_body(i, carry):
        zdeg[i, :] = zero16
        return carry
    lax.fori_loop(0, 16, zd_body, 0)

    def ones_body(i, carry):
        ones_v[i, :] = one16
        return carry
    lax.fori_loop(0, C2, ones_body, 0)

    row0 = s * RPB
    ncopy = jnp.where(s == NT - 1, (RPB + 16) // 16, RPB // 16)

    def zinit(i, carry):
        pltpu.sync_copy(zdeg, deg_sh.at[pl.ds(row0 + i * 16, 16)])
        return carry
    lax.fori_loop(0, ncopy, zinit, 0)
    plsc.subcore_barrier()

    w = c * NT + s
    bufs = ((idx20, urows0, vrows0, wrows0, sem0, isem0),
            (idx21, urows1, vrows1, wrows1, sem1, isem1))

    def prefetch(k, b):
        idx2_v, _, _, _, _, isem = b
        pltpu.async_copy(eidx_hbm.at[w + 32 * k], idx2_v, isem)

    def fire(k, b):
        idx2_v, ur, vr, wr, sem, isem = b
        cid = w + 32 * k
        base = cid * C2
        pltpu.make_async_copy(eidx_hbm.at[cid], idx2_v, isem).wait()
        pltpu.async_copy(wu_hbm.at[idx2_v.at[0]], ur, sem)
        pltpu.async_copy(wv_hbm.at[idx2_v.at[1]], vr, sem)
        pltpu.async_copy(ew_hbm.at[pl.ds(base, C2)], wr, sem)

    def consume(k, b):
        idx2_v, ur, vr, wr, sem, isem = b
        cid = w + 32 * k
        base = cid * C2
        pltpu.make_async_copy(wu_hbm.at[idx2_v.at[0]], ur, sem).wait()
        pltpu.make_async_copy(wv_hbm.at[idx2_v.at[1]], vr, sem).wait()
        pltpu.make_async_copy(ew_hbm.at[pl.ds(base, C2)], wr, sem).wait()

        def erow(i, carry2):
            wr[i, :] = jnp.maximum(wr[i, :] + ur[i, :] + vr[i, :], 0.0)
            return carry2
        lax.fori_loop(0, C2, erow, 0)
        pltpu.sync_copy(wr, enew_hbm.at[pl.ds(base, C2)])
        pltpu.sync_copy(ones_v, deg_sh.at[idx2_v.at[1]], add=True)

    prefetch(0, bufs[0])
    prefetch(1, bufs[1])
    fire(0, bufs[0])
    PAIRS = (CH2 - 1) // 2

    def pair(i, carry):
        k0 = 2 * i
        fire(k0 + 1, bufs[1])
        consume(k0, bufs[0])
        prefetch(k0 + 2, bufs[0])

        @pl.when(i < PAIRS - 1)
        def _():
            fire(k0 + 2, bufs[0])
        consume(k0 + 1, bufs[1])

        @pl.when(i < PAIRS - 1)
        def _():
            prefetch(k0 + 3, bufs[1])
        return carry
    lax.fori_loop(0, PAIRS, pair, 0)

    for k in range(2 * PAIRS, CH2):
        fire(k, bufs[k % 2])
        consume(k, bufs[k % 2])

    @pl.when(w < NCH2 - 32 * CH2)
    def _():
        prefetch(CH2, bufs[CH2 % 2])
        fire(CH2, bufs[CH2 % 2])
        consume(CH2, bufs[CH2 % 2])
    plsc.subcore_barrier()

    @pl.when(s < NT - 1)
    def _():
        pltpu.sync_copy(deg_sh.at[pl.ds(row0, RPB)],
                        deg_hbm.at[pl.ds(c * N + row0, RPB)])

    @pl.when(s == NT - 1)
    def _():
        last0 = (NT - 1) * RPB
        nlast = N - last0
        pltpu.sync_copy(deg_sh.at[pl.ds(last0, nlast)],
                        deg_hbm.at[pl.ds(c * N + last0, nlast)])


_sc_edge_new = functools.partial(
    pl.kernel,
    out_type=(
        jax.ShapeDtypeStruct((E, DE), jnp.float32),
        jax.ShapeDtypeStruct((2 * N, DE), jnp.float32),
    ),
    mesh=plsc.VectorSubcoreMesh(core_axis_name="c", subcore_axis_name="s"),
    compiler_params=pltpu.CompilerParams(use_tc_tiling_on_sc=False),
    scratch_types=(
        pltpu.VMEM_SHARED((N, DE), jnp.float32),
        pltpu.VMEM((2, C2), jnp.int32),
        pltpu.VMEM((C2, DE), jnp.float32),
        pltpu.VMEM((C2, DE), jnp.float32),
        pltpu.VMEM((C2, DE), jnp.float32),
        pltpu.SemaphoreType.DMA,
        pltpu.SemaphoreType.DMA,
        pltpu.VMEM((2, C2), jnp.int32),
        pltpu.VMEM((C2, DE), jnp.float32),
        pltpu.VMEM((C2, DE), jnp.float32),
        pltpu.VMEM((C2, DE), jnp.float32),
        pltpu.SemaphoreType.DMA,
        pltpu.SemaphoreType.DMA,
        pltpu.VMEM((C2, DE), jnp.float32),
        pltpu.VMEM((16, DE), jnp.float32),
    ),
)(_sc2_body)


def kernel(h, e, edge_index, P_w, P_b, Q_w, Q_b, W_w, W_b):
    src = edge_index[0].astype(jnp.int32)
    tgt = edge_index[1].astype(jnp.int32)

    phT = P_w[:, :D].T
    peT = P_w[:, D:].T
    weT = W_w[:, :DE].T
    wuT = W_w[:, DE:DE + D].T
    wvT = W_w[:, DE + D:].T
    qhT = Q_w[:, :D].T
    qaT = Q_w[:, D:].T

    ei = jnp.stack([src, tgt])
    eidx1 = ei.reshape(2, E // C, C).transpose(1, 0, 2)
    eidx2 = ei.reshape(2, E // C2, C2).transpose(1, 0, 2)

    hp, wu, wv = _tc_node_pre(h, phT, wuT, wvT)
    ep, ew = _tc_edge_pre(e, peT, P_b.reshape(1, D), weT, W_b.reshape(1, DE))

    e_new, deg2 = _sc_edge_new(eidx2, wu, wv, ew)
    agg = _sc_aggregate(hp.reshape(2 * N, H), ep.reshape(2 * E, H), eidx1)

    h_new = _tc_node_out(h, agg, deg2, qhT, qaT, Q_b.reshape(1, D))
    return (h_new, e_new)


# speedup vs baseline: 1.0234x; 1.0234x over previous
import functools

import jax
import jax.numpy as jnp
from jax import lax
from jax.experimental import pallas as pl
from jax.experimental.pallas import tpu as pltpu
from jax.experimental.pallas import tpu_sc as plsc

N = 10000
E = 160000
D = 256
DE = 16
H = 128

NT = 16
EPT = E // NT
C = 80
CH = EPT // C
RPB = 624

C2 = 128
NCH2 = E // C2
CH2 = NCH2 // 32

_NODE_BLK = 2000
_EDGE_BLK = 4000


def _tc_node_pre_body(h_ref, phT_ref, wuT_ref, wvT_ref, hp_ref, wu_ref, wv_ref):
    hblk = h_ref[...]
    hp = jnp.dot(hblk, phT_ref[...], preferred_element_type=jnp.float32)
    hp_ref[0] = hp[:, :H]
    hp_ref[1] = hp[:, H:]
    wu_ref[...] = jnp.dot(hblk, wuT_ref[...], preferred_element_type=jnp.float32)
    wv_ref[...] = jnp.dot(hblk, wvT_ref[...], preferred_element_type=jnp.float32)


def _tc_node_pre(h, phT, wuT, wvT):
    nb = N // _NODE_BLK
    return pl.pallas_call(
        _tc_node_pre_body,
        grid=(nb,),
        in_specs=[
            pl.BlockSpec((_NODE_BLK, D), lambda i: (i, 0)),
            pl.BlockSpec((D, D), lambda i: (0, 0)),
            pl.BlockSpec((D, DE), lambda i: (0, 0)),
            pl.BlockSpec((D, DE), lambda i: (0, 0)),
        ],
        out_specs=[
            pl.BlockSpec((2, _NODE_BLK, H), lambda i: (0, i, 0)),
            pl.BlockSpec((_NODE_BLK, DE), lambda i: (i, 0)),
            pl.BlockSpec((_NODE_BLK, DE), lambda i: (i, 0)),
        ],
        out_shape=[
            jax.ShapeDtypeStruct((2, N, H), jnp.float32),
            jax.ShapeDtypeStruct((N, DE), jnp.float32),
            jax.ShapeDtypeStruct((N, DE), jnp.float32),
        ],
    )(h, phT, wuT, wvT)


def _tc_edge_pre_body(e_ref, peT_ref, pb_ref, ep_ref):
    ep = (jnp.dot(e_ref[...], peT_ref[...], preferred_element_type=jnp.float32)
          + pb_ref[...])
    ep_ref[0] = ep[:, :H]
    ep_ref[1] = ep[:, H:]


def _tc_edge_pre(e, peT, pb):
    nb = E // _EDGE_BLK
    return pl.pallas_call(
        _tc_edge_pre_body,
        grid=(nb,),
        in_specs=[
            pl.BlockSpec((_EDGE_BLK, DE), lambda i: (i, 0)),
            pl.BlockSpec((DE, D), lambda i: (0, 0)),
            pl.BlockSpec((1, D), lambda i: (0, 0)),
        ],
        out_specs=pl.BlockSpec((2, _EDGE_BLK, H), lambda i: (0, i, 0)),
        out_shape=jax.ShapeDtypeStruct((2, E, H), jnp.float32),
    )(e, peT, pb)


def _tc_edge_w_body(e_ref, weT_ref, wb_ref, ew_ref):
    ew_ref[...] = (jnp.dot(e_ref[...], weT_ref[...],
                           preferred_element_type=jnp.float32) + wb_ref[...])


def _tc_edge_w(e, weT, wb):
    nb = E // _EDGE_BLK
    return pl.pallas_call(
        _tc_edge_w_body,
        grid=(nb,),
        in_specs=[
            pl.BlockSpec((_EDGE_BLK, DE), lambda i: (i, 0)),
            pl.BlockSpec((DE, DE), lambda i: (0, 0)),
            pl.BlockSpec((1, DE), lambda i: (0, 0)),
        ],
        out_specs=pl.BlockSpec((_EDGE_BLK, DE), lambda i: (i, 0)),
        out_shape=jax.ShapeDtypeStruct((E, DE), jnp.float32),
    )(e, weT, wb)


def _tc_node_out_body(h_ref, a0_ref, a1_ref, d0_ref, d1_ref, qhT_ref, qaT_ref,
                      qb_ref, out_ref):
    agg = jnp.concatenate([a0_ref[...], a1_ref[...]], axis=-1)
    deg = d0_ref[:, 0:1] + d1_ref[:, 0:1]
    degc = jnp.maximum(deg, 1.0)
    aggn = agg / degc
    acc = (jnp.dot(h_ref[...], qhT_ref[...], preferred_element_type=jnp.float32)
           + jnp.dot(aggn, qaT_ref[...], preferred_element_type=jnp.float32)
           + qb_ref[...])
    out_ref[...] = jnp.maximum(acc, 0.0)


def _tc_node_out(h, agg, deg2, qhT, qaT, qb):
    nb = N // _NODE_BLK
    return pl.pallas_call(
        _tc_node_out_body,
        grid=(nb,),
        in_specs=[
            pl.BlockSpec((_NODE_BLK, D), lambda i: (i, 0)),
            pl.BlockSpec((_NODE_BLK, H), lambda i: (i, 0)),
            pl.BlockSpec((_NODE_BLK, H), lambda i, nb=nb: (i + nb, 0)),
            pl.BlockSpec((_NODE_BLK, DE), lambda i: (i, 0)),
            pl.BlockSpec((_NODE_BLK, DE), lambda i, nb=nb: (i + nb, 0)),
            pl.BlockSpec((D, D), lambda i: (0, 0)),
            pl.BlockSpec((D, D), lambda i: (0, 0)),
            pl.BlockSpec((1, D), lambda i: (0, 0)),
        ],
        out_specs=pl.BlockSpec((_NODE_BLK, D), lambda i: (i, 0)),
        out_shape=jax.ShapeDtypeStruct((N, D), jnp.float32),
    )(h, agg, agg, deg2, deg2, qhT, qaT, qb)


def _sc1_body(hp_hbm, ep_hbm, eidx_hbm, agg_hbm,
              agg_sh,
              idx20, src20, hrows0, erows0, sem0, isem0,
              idx21, src21, hrows1, erows1, sem1, isem1,
              zbuf):
    c = lax.axis_index("c")
    s = lax.axis_index("s")
    zero16 = jnp.zeros((16,), jnp.float32)

    def zb_body(i, carry):
        for j in range(H // 16):
            zbuf[i, pl.ds(j * 16, 16)] = zero16
        return carry
    lax.fori_loop(0, 16, zb_body, 0)

    row0 = s * RPB
    ncopy = jnp.where(s == NT - 1, (RPB + 16) // 16, RPB // 16)

    def zinit(i, carry):
        pltpu.sync_copy(zbuf, agg_sh.at[pl.ds(row0 + i * 16, 16)])
        return carry
    lax.fori_loop(0, ncopy, zinit, 0)
    plsc.subcore_barrier()

    tile_base = s * EPT
    hp_off = c * N
    bufs = ((idx20, src20, hrows0, erows0, sem0, isem0),
            (idx21, src21, hrows1, erows1, sem1, isem1))

    def prefetch(k, b):
        idx2_v, _, _, _, _, isem = b
        pltpu.async_copy(eidx_hbm.at[s * CH + k], idx2_v, isem)

    def fire(k, b):
        idx2_v, src2_v, hr, er, sem, isem = b
        base = tile_base + k * C
        pltpu.make_async_copy(eidx_hbm.at[s * CH + k], idx2_v, isem).wait()
        for j in range(C // 16):
            sl = pl.ds(j * 16, 16)
            src2_v[sl] = idx2_v[0, sl] + hp_off
        pltpu.async_copy(hp_hbm.at[src2_v], hr, sem)
        pltpu.async_copy(ep_hbm.at[pl.ds(c * E + base, C)], er, sem)

    def consume(k, b):
        idx2_v, src2_v, hr, er, sem, isem = b
        base = tile_base + k * C
        pltpu.make_async_copy(hp_hbm.at[src2_v], hr, sem).wait()
        pltpu.make_async_copy(ep_hbm.at[pl.ds(c * E + base, C)], er, sem).wait()

        def row(i, carry2):
            for jj in range(4):
                for j in range(H // 16):
                    sl = pl.ds(j * 16, 16)
                    hr[4 * i + jj, sl] = jnp.maximum(
                        hr[4 * i + jj, sl] + er[4 * i + jj, sl], 0.0)
            return carry2
        lax.fori_loop(0, C // 4, row, 0)
        pltpu.sync_copy(hr, agg_sh.at[idx2_v.at[1]], add=True)

    prefetch(0, bufs[0])
    prefetch(1, bufs[1])
    fire(0, bufs[0])
    PAIRS = (CH - 1) // 2

    def pair(i, carry):
        k0 = 2 * i
        fire(k0 + 1, bufs[1])
        consume(k0, bufs[0])
        prefetch(k0 + 2, bufs[0])

        @pl.when(i < PAIRS - 1)
        def _():
            fire(k0 + 2, bufs[0])
        consume(k0 + 1, bufs[1])

        @pl.when(i < PAIRS - 1)
        def _():
            prefetch(k0 + 3, bufs[1])
        return carry
    lax.fori_loop(0, PAIRS, pair, 0)

    for k in range(2 * PAIRS, CH):
        fire(k, bufs[k % 2])
        consume(k, bufs[k % 2])
    plsc.subcore_barrier()

    @pl.when(s < NT - 1)
    def _():
        pltpu.sync_copy(agg_sh.at[pl.ds(row0, RPB)],
                        agg_hbm.at[pl.ds(c * N + row0, RPB)])

    @pl.when(s == NT - 1)
    def _():
        last0 = (NT - 1) * RPB
        nlast = N - last0
        pltpu.sync_copy(agg_sh.at[pl.ds(last0, nlast)],
                        agg_hbm.at[pl.ds(c * N + last0, nlast)])


_sc_aggregate = functools.partial(
    pl.kernel,
    out_type=jax.ShapeDtypeStruct((2 * N, H), jnp.float32),
    mesh=plsc.VectorSubcoreMesh(core_axis_name="c", subcore_axis_name="s"),
    compiler_params=pltpu.CompilerParams(use_tc_tiling_on_sc=False),
    scratch_types=(
        pltpu.VMEM_SHARED((N, H), jnp.float32),
        pltpu.VMEM((2, C), jnp.int32),
        pltpu.VMEM((C,), jnp.int32),
        pltpu.VMEM((C, H), jnp.float32),
        pltpu.VMEM((C, H), jnp.float32),
        pltpu.SemaphoreType.DMA,
        pltpu.SemaphoreType.DMA,
        pltpu.VMEM((2, C), jnp.int32),
        pltpu.VMEM((C,), jnp.int32),
        pltpu.VMEM((C, H), jnp.float32),
        pltpu.VMEM((C, H), jnp.float32),
        pltpu.SemaphoreType.DMA,
        pltpu.SemaphoreType.DMA,
        pltpu.VMEM((16, H), jnp.float32),
    ),
)(_sc1_body)


def _sc2_body(eidx_hbm, wu_hbm, wv_hbm, ew_hbm,
              enew_hbm, deg_hbm,
              deg_sh,
              idx20, urows0, vrows0, wrows0, sem0, isem0,
              idx21, urows1, vrows1, wrows1, sem1, isem1,
              ones_v, zdeg):
    c = lax.axis_index("c")
    s = lax.axis_index("s")
    zero16 = jnp.zeros((16,), jnp.float32)
    one16 = jnp.full((16,), 1.0, jnp.float32)

    def zd_body(i, carry):
        zdeg[i, :] = zero16
        return carry
    lax.fori_loop(0, 16, zd_body, 0)

    def ones_body(i, carry):
        ones_v[i, :] = one16
        return carry
    lax.fori_loop(0, C2, ones_body, 0)

    row0 = s * RPB
    ncopy = jnp.where(s == NT - 1, (RPB + 16) // 16, RPB // 16)

    def zinit(i, carry):
        pltpu.sync_copy(zdeg, deg_sh.at[pl.ds(row0 + i * 16, 16)])
        return carry
    lax.fori_loop(0, ncopy, zinit, 0)
    plsc.subcore_barrier()

    w = c * NT + s
    bufs = ((idx20, urows0, vrows0, wrows0, sem0, isem0),
            (idx21, urows1, vrows1, wrows1, sem1, isem1))

    def prefetch(k, b):
        idx2_v, _, _, _, _, isem = b
        pltpu.async_copy(eidx_hbm.at[w + 32 * k], idx2_v, isem)

    def fire(k, b):
        idx2_v, ur, vr, wr, sem, isem = b
        cid = w + 32 * k
        base = cid * C2
        pltpu.make_async_copy(eidx_hbm.at[cid], idx2_v, isem).wait()
        pltpu.async_copy(wu_hbm.at[idx2_v.at[0]], ur, sem)
        pltpu.async_copy(wv_hbm.at[idx2_v.at[1]], vr, sem)
        pltpu.async_copy(ew_hbm.at[pl.ds(base, C2)], wr, sem)

    def consume(k, b):
        idx2_v, ur, vr, wr, sem, isem = b
        cid = w + 32 * k
        base = cid * C2
        pltpu.make_async_copy(wu_hbm.at[idx2_v.at[0]], ur, sem).wait()
        pltpu.make_async_copy(wv_hbm.at[idx2_v.at[1]], vr, sem).wait()
        pltpu.make_async_copy(ew_hbm.at[pl.ds(base, C2)], wr, sem).wait()

        def erow(i, carry2):
            wr[i, :] = jnp.maximum(wr[i, :] + ur[i, :] + vr[i, :], 0.0)
            return carry2
        lax.fori_loop(0, C2, erow, 0)
        pltpu.sync_copy(wr, enew_hbm.at[pl.ds(base, C2)])
        pltpu.sync_copy(ones_v, deg_sh.at[idx2_v.at[1]], add=True)

    prefetch(0, bufs[0])
    prefetch(1, bufs[1])
    fire(0, bufs[0])
    PAIRS = (CH2 - 1) // 2

    def pair(i, carry):
        k0 = 2 * i
        fire(k0 + 1, bufs[1])
        consume(k0, bufs[0])
        prefetch(k0 + 2, bufs[0])

        @pl.when(i < PAIRS - 1)
        def _():
            fire(k0 + 2, bufs[0])
        consume(k0 + 1, bufs[1])

        @pl.when(i < PAIRS - 1)
        def _():
            prefetch(k0 + 3, bufs[1])
        return carry
    lax.fori_loop(0, PAIRS, pair, 0)

    for k in range(2 * PAIRS, CH2):
        fire(k, bufs[k % 2])
        consume(k, bufs[k % 2])

    @pl.when(w < NCH2 - 32 * CH2)
    def _():
        prefetch(CH2, bufs[CH2 % 2])
        fire(CH2, bufs[CH2 % 2])
        consume(CH2, bufs[CH2 % 2])
    plsc.subcore_barrier()

    @pl.when(s < NT - 1)
    def _():
        pltpu.sync_copy(deg_sh.at[pl.ds(row0, RPB)],
                        deg_hbm.at[pl.ds(c * N + row0, RPB)])

    @pl.when(s == NT - 1)
    def _():
        last0 = (NT - 1) * RPB
        nlast = N - last0
        pltpu.sync_copy(deg_sh.at[pl.ds(last0, nlast)],
                        deg_hbm.at[pl.ds(c * N + last0, nlast)])


_sc_edge_new = functools.partial(
    pl.kernel,
    out_type=(
        jax.ShapeDtypeStruct((E, DE), jnp.float32),
        jax.ShapeDtypeStruct((2 * N, DE), jnp.float32),
    ),
    mesh=plsc.VectorSubcoreMesh(core_axis_name="c", subcore_axis_name="s"),
    compiler_params=pltpu.CompilerParams(use_tc_tiling_on_sc=False),
    scratch_types=(
        pltpu.VMEM_SHARED((N, DE), jnp.float32),
        pltpu.VMEM((2, C2), jnp.int32),
        pltpu.VMEM((C2, DE), jnp.float32),
        pltpu.VMEM((C2, DE), jnp.float32),
        pltpu.VMEM((C2, DE), jnp.float32),
        pltpu.SemaphoreType.DMA,
        pltpu.SemaphoreType.DMA,
        pltpu.VMEM((2, C2), jnp.int32),
        pltpu.VMEM((C2, DE), jnp.float32),
        pltpu.VMEM((C2, DE), jnp.float32),
        pltpu.VMEM((C2, DE), jnp.float32),
        pltpu.SemaphoreType.DMA,
        pltpu.SemaphoreType.DMA,
        pltpu.VMEM((C2, DE), jnp.float32),
        pltpu.VMEM((16, DE), jnp.float32),
    ),
)(_sc2_body)


def kernel(h, e, edge_index, P_w, P_b, Q_w, Q_b, W_w, W_b):
    src = edge_index[0].astype(jnp.int32)
    tgt = edge_index[1].astype(jnp.int32)

    phT = P_w[:, :D].T
    peT = P_w[:, D:].T
    weT = W_w[:, :DE].T
    wuT = W_w[:, DE:DE + D].T
    wvT = W_w[:, DE + D:].T
    qhT = Q_w[:, :D].T
    qaT = Q_w[:, D:].T

    ei = jnp.stack([src, tgt])
    eidx1 = ei.reshape(2, E // C, C).transpose(1, 0, 2)
    eidx2 = ei.reshape(2, E // C2, C2).transpose(1, 0, 2)

    hp, wu, wv = _tc_node_pre(h, phT, wuT, wvT)
    ew = _tc_edge_w(e, weT, W_b.reshape(1, DE))
    ep = _tc_edge_pre(e, peT, P_b.reshape(1, D))

    e_new, deg2 = _sc_edge_new(eidx2, wu, wv, ew)
    agg = _sc_aggregate(hp.reshape(2 * N, H), ep.reshape(2 * E, H), eidx1)

    h_new = _tc_node_out(h, agg, deg2, qhT, qaT, Q_b.reshape(1, D))
    return (h_new, e_new)
